# Initial kernel scaffold; baseline (speedup 1.0000x reference)
import jax
import jax.numpy as jnp
from jax.experimental import pallas as pl


def kernel(x, edge_index, edge_attr, W_node, W_edge, W_gcn, b_gcn, W_out):
    raise NotImplementedError("write your pallas kernel here")


# speedup vs baseline: 3.9700x; 3.9700x over previous
import functools

import jax
import jax.numpy as jnp
from jax import lax
from jax.experimental import pallas as pl
from jax.experimental.pallas import tpu as pltpu
from jax.experimental.pallas import tpu_sc as plsc

N = 50000
E = 800000
H = 64
HH = 32
L = 4

NC = 2
NS = 16

CHUNK = 128
EPT = E // NS
NFULL = EPT // CHUNK
TAIL = EPT - NFULL * CHUNK
ZCH = 500
NZCH = N // ZCH
CPCH = 1000
NCP = N // CPCH


def _sc_segment_sum_body(h2, src, dst, out, agg, rows, rows_t, sidx, didx,
                         sidx_t, didx_t, zrows, gsem):
    cid = lax.axis_index("c")
    sid = lax.axis_index("s")

    def zero_row(r, _):
        z = jnp.zeros((16,), jnp.float32)
        zrows[r, pl.ds(0, 16)] = z
        zrows[r, pl.ds(16, 16)] = z
        return 0

    lax.fori_loop(0, ZCH, zero_row, 0)

    def zero_dma(j, _):
        idx = sid + j * NS

        @pl.when(idx < NZCH)
        def _():
            pltpu.sync_copy(zrows, agg.at[pl.ds(idx * ZCH, ZCH)])

        return 0

    lax.fori_loop(0, (NZCH + NS - 1) // NS, zero_dma, 0)

    plsc.subcore_barrier()

    hhalf = h2.at[cid]
    ebase = sid * EPT

    def edge_chunk(it, _):
        base = ebase + it * CHUNK
        pltpu.sync_copy(src.at[pl.ds(base, CHUNK)], sidx)
        pltpu.sync_copy(dst.at[pl.ds(base, CHUNK)], didx)
        pltpu.async_copy(hhalf.at[sidx], rows, gsem).wait()
        pltpu.sync_copy(rows, agg.at[didx], add=True)
        return 0

    lax.fori_loop(0, NFULL, edge_chunk, 0)

    tbase = ebase + NFULL * CHUNK
    pltpu.sync_copy(src.at[pl.ds(tbase, TAIL)], sidx_t)
    pltpu.sync_copy(dst.at[pl.ds(tbase, TAIL)], didx_t)
    pltpu.async_copy(hhalf.at[sidx_t], rows_t, gsem).wait()
    pltpu.sync_copy(rows_t, agg.at[didx_t], add=True)

    plsc.subcore_barrier()

    def copy_out(j, _):
        idx = sid + j * NS

        @pl.when(idx < NCP)
        def _():
            pltpu.sync_copy(agg.at[pl.ds(idx * CPCH, CPCH)],
                            out.at[cid].at[pl.ds(idx * CPCH, CPCH)])

        return 0

    lax.fori_loop(0, (NCP + NS - 1) // NS, copy_out, 0)


_sc_segment_sum = functools.partial(
    pl.kernel,
    out_type=jax.ShapeDtypeStruct((NC, N, HH), jnp.float32),
    mesh=plsc.VectorSubcoreMesh(core_axis_name="c", subcore_axis_name="s",
                                num_cores=NC, num_subcores=NS),
    scratch_types=[
        pltpu.VMEM_SHARED((N, HH), jnp.float32),
        pltpu.VMEM((CHUNK, HH), jnp.float32),
        pltpu.VMEM((TAIL, HH), jnp.float32),
        pltpu.VMEM((CHUNK,), jnp.int32),
        pltpu.VMEM((CHUNK,), jnp.int32),
        pltpu.VMEM((TAIL,), jnp.int32),
        pltpu.VMEM((TAIL,), jnp.int32),
        pltpu.VMEM((ZCH, HH), jnp.float32),
        pltpu.SemaphoreType.DMA,
    ],
    compiler_params=pltpu.CompilerParams(use_tc_tiling_on_sc=False),
)(_sc_segment_sum_body)


def sc_segment_sum(h2, src, dst):
    return _sc_segment_sum(h2, src, dst)


RBLK = 2000
NSTEPS = N // RBLK


def _embed_body(x_ref, w_ref, out_ref):
    y = jnp.dot(x_ref[...], w_ref[...], preferred_element_type=jnp.float32)
    out_ref[0] = y[:, :HH]
    out_ref[1] = y[:, HH:]


def tc_embed(x_pad, w_pad):
    return pl.pallas_call(
        _embed_body,
        grid=(NSTEPS,),
        in_specs=[
            pl.BlockSpec((RBLK, H), lambda i: (i, 0)),
            pl.BlockSpec((H, H), lambda i: (0, 0)),
        ],
        out_specs=pl.BlockSpec((NC, RBLK, HH), lambda i: (0, i, 0)),
        out_shape=jax.ShapeDtypeStruct((NC, N, HH), jnp.float32),
    )(x_pad, w_pad)


def _layer_body(agg_ref, w_ref, b_ref, out_ref):
    a = agg_ref[...]
    w = w_ref[...]
    y = (jnp.dot(a[0], w[:HH, :], preferred_element_type=jnp.float32)
         + jnp.dot(a[1], w[HH:, :], preferred_element_type=jnp.float32)
         + b_ref[...])
    y = jnp.maximum(y, 0.0)
    out_ref[0] = y[:, :HH]
    out_ref[1] = y[:, HH:]


def tc_layer(agg2, w, b_row):
    return pl.pallas_call(
        _layer_body,
        grid=(NSTEPS,),
        in_specs=[
            pl.BlockSpec((NC, RBLK, HH), lambda i: (0, i, 0)),
            pl.BlockSpec((H, H), lambda i: (0, 0)),
            pl.BlockSpec((1, H), lambda i: (0, 0)),
        ],
        out_specs=pl.BlockSpec((NC, RBLK, HH), lambda i: (0, i, 0)),
        out_shape=jax.ShapeDtypeStruct((NC, N, HH), jnp.float32),
    )(agg2, w, b_row)


def _final_body(agg_ref, w_ref, b_ref, wout_ref, out_ref, acc_ref):
    i = pl.program_id(0)

    @pl.when(i == 0)
    def _():
        acc_ref[...] = jnp.zeros_like(acc_ref)

    a = agg_ref[...]
    w = w_ref[...]
    y = (jnp.dot(a[0], w[:HH, :], preferred_element_type=jnp.float32)
         + jnp.dot(a[1], w[HH:, :], preferred_element_type=jnp.float32)
         + b_ref[...])
    y = jnp.maximum(y, 0.0)
    acc_ref[...] += jnp.sum(y, axis=0, keepdims=True)

    @pl.when(i == NSTEPS - 1)
    def _():
        out_ref[...] = jnp.sum(acc_ref[...] * wout_ref[...]).reshape(1, 1)


def tc_final(agg2, w, b_row, wout_row):
    return pl.pallas_call(
        _final_body,
        grid=(NSTEPS,),
        in_specs=[
            pl.BlockSpec((NC, RBLK, HH), lambda i: (0, i, 0)),
            pl.BlockSpec((H, H), lambda i: (0, 0)),
            pl.BlockSpec((1, H), lambda i: (0, 0)),
            pl.BlockSpec((1, H), lambda i: (0, 0)),
        ],
        out_specs=pl.BlockSpec((1, 1), lambda i: (0, 0)),
        out_shape=jax.ShapeDtypeStruct((1, 1), jnp.float32),
        scratch_shapes=[pltpu.VMEM((1, H), jnp.float32)],
    )(agg2, w, b_row, wout_row)


def kernel(x, edge_index, edge_attr, W_node, W_edge, W_gcn, b_gcn, W_out):
    del edge_attr, W_edge
    d_in = x.shape[1]
    x_pad = jnp.pad(x, ((0, 0), (0, H - d_in)))
    w_pad = jnp.pad(W_node, ((0, H - d_in), (0, 0)))
    src = edge_index[0]
    dst = edge_index[1]

    h2 = tc_embed(x_pad, w_pad)
    for i in range(L):
        agg2 = sc_segment_sum(h2, src, dst)
        b_row = b_gcn[i].reshape(1, H)
        if i < L - 1:
            h2 = tc_layer(agg2, W_gcn[i], b_row)
        else:
            out = tc_final(agg2, W_gcn[i], b_row, W_out.reshape(1, H))
    return out


# speedup vs baseline: 11.5048x; 2.8979x over previous
import functools

import jax
import jax.numpy as jnp
from jax import lax
from jax.experimental import pallas as pl
from jax.experimental.pallas import tpu as pltpu
from jax.experimental.pallas import tpu_sc as plsc

N = 50000
E = 800000
H = 64
HH = 32
L = 4

NC = 2
NS = 16

CH = 80
CPS = 25
EPT = E // NS
NCHUNK = EPT // CH
NSUP = NCHUNK // CPS
KR = 5
DP = 2
ZCH = 100
NZCH = N // ZCH
CPCH = 1000
NCP = N // CPCH


def _sc_segment_sum_body(h2, src4, dst4, out, agg, rows, sidx, didx, zrows,
                         isem, gsem, ssem):
    cid = lax.axis_index("c")
    sid = lax.axis_index("s")

    pltpu.async_copy(src4.at[sid].at[pl.ds(0, CPS)], sidx.at[0], isem.at[0])
    pltpu.async_copy(dst4.at[sid].at[pl.ds(0, CPS)], didx.at[0], isem.at[0])

    def zero_row(r, _):
        z = jnp.zeros((16,), jnp.float32)
        zrows[r, pl.ds(0, 16)] = z
        zrows[r, pl.ds(16, 16)] = z
        return 0

    lax.fori_loop(0, ZCH, zero_row, 0)

    def zero_dma(j, _):
        idx = sid + j * NS

        @pl.when(idx < NZCH)
        def _():
            pltpu.sync_copy(zrows, agg.at[pl.ds(idx * ZCH, ZCH)])

        return 0

    lax.fori_loop(0, (NZCH + NS - 1) // NS, zero_dma, 0)

    plsc.subcore_barrier()

    hhalf = h2.at[cid]

    def wait_gather(j):
        pltpu.make_async_copy(hhalf.at[pl.ds(0, CH)], rows.at[j], gsem.at[j]).wait()

    def wait_scatter(j):
        pltpu.make_async_copy(rows.at[j], agg.at[pl.ds(0, CH)], ssem.at[j]).wait()

    def pipeline_step(c, _):
        @pl.when(c < NCHUNK)
        def _():
            sup = c // CPS
            j = lax.rem(c, CPS)
            tb = lax.rem(sup, 3)

            @pl.when(j == 0)
            def _():
                @pl.when(sup < NSUP - 1)
                def _():
                    tn = lax.rem(sup + 1, 3)
                    base = (sup + 1) * CPS
                    pltpu.async_copy(src4.at[sid].at[pl.ds(base, CPS)],
                                     sidx.at[tn], isem.at[tn])
                    pltpu.async_copy(dst4.at[sid].at[pl.ds(base, CPS)],
                                     didx.at[tn], isem.at[tn])

                pltpu.make_async_copy(src4.at[sid].at[pl.ds(0, CPS)],
                                      sidx.at[tb], isem.at[tb]).wait()
                pltpu.make_async_copy(dst4.at[sid].at[pl.ds(0, CPS)],
                                      didx.at[tb], isem.at[tb]).wait()

            b = lax.rem(c, KR)

            @pl.when(c >= KR)
            def _():
                wait_scatter(b)

            pltpu.async_copy(hhalf.at[sidx.at[tb, j]], rows.at[b], gsem.at[b])

        @pl.when(c >= DP)
        def _():
            cc = c - DP
            sup2 = cc // CPS
            j2 = lax.rem(cc, CPS)
            tb2 = lax.rem(sup2, 3)
            b2 = lax.rem(cc, KR)
            wait_gather(b2)
            pltpu.async_copy(rows.at[b2], agg.at[didx.at[tb2, j2]],
                             ssem.at[b2], add=True)

        return 0

    lax.fori_loop(0, NCHUNK + DP, pipeline_step, 0)

    def drain(j, _):
        wait_scatter(j)
        return 0

    lax.fori_loop(0, KR, drain, 0)

    plsc.subcore_barrier()

    def copy_out(j, _):
        idx = sid + j * NS

        @pl.when(idx < NCP)
        def _():
            pltpu.sync_copy(agg.at[pl.ds(idx * CPCH, CPCH)],
                            out.at[cid].at[pl.ds(idx * CPCH, CPCH)])

        return 0

    lax.fori_loop(0, (NCP + NS - 1) // NS, copy_out, 0)


_sc_segment_sum = functools.partial(
    pl.kernel,
    out_type=jax.ShapeDtypeStruct((NC, N, HH), jnp.float32),
    mesh=plsc.VectorSubcoreMesh(core_axis_name="c", subcore_axis_name="s",
                                num_cores=NC, num_subcores=NS),
    scratch_types=[
        pltpu.VMEM_SHARED((N, HH), jnp.float32),
        pltpu.VMEM((KR, CH, HH), jnp.float32),
        pltpu.VMEM((3, CPS, CH), jnp.int32),
        pltpu.VMEM((3, CPS, CH), jnp.int32),
        pltpu.VMEM((ZCH, HH), jnp.float32),
        pltpu.SemaphoreType.DMA((3,)),
        pltpu.SemaphoreType.DMA((KR,)),
        pltpu.SemaphoreType.DMA((KR,)),
    ],
    compiler_params=pltpu.CompilerParams(use_tc_tiling_on_sc=False),
)(_sc_segment_sum_body)


def sc_segment_sum(h2, src4, dst4):
    return _sc_segment_sum(h2, src4, dst4)


RBLK = 2000
NSTEPS = N // RBLK


def _embed_body(x_ref, w_ref, out_ref):
    y = jnp.dot(x_ref[...], w_ref[...], preferred_element_type=jnp.float32)
    out_ref[0] = y[:, :HH]
    out_ref[1] = y[:, HH:]


def tc_embed(x_pad, w_pad):
    return pl.pallas_call(
        _embed_body,
        grid=(NSTEPS,),
        in_specs=[
            pl.BlockSpec((RBLK, H), lambda i: (i, 0)),
            pl.BlockSpec((H, H), lambda i: (0, 0)),
        ],
        out_specs=pl.BlockSpec((NC, RBLK, HH), lambda i: (0, i, 0)),
        out_shape=jax.ShapeDtypeStruct((NC, N, HH), jnp.float32),
    )(x_pad, w_pad)


def _layer_body(agg_ref, w_ref, b_ref, out_ref):
    a = agg_ref[...]
    w = w_ref[...]
    y = (jnp.dot(a[0], w[:HH, :], preferred_element_type=jnp.float32)
         + jnp.dot(a[1], w[HH:, :], preferred_element_type=jnp.float32)
         + b_ref[...])
    y = jnp.maximum(y, 0.0)
    out_ref[0] = y[:, :HH]
    out_ref[1] = y[:, HH:]


def tc_layer(agg2, w, b_row):
    return pl.pallas_call(
        _layer_body,
        grid=(NSTEPS,),
        in_specs=[
            pl.BlockSpec((NC, RBLK, HH), lambda i: (0, i, 0)),
            pl.BlockSpec((H, H), lambda i: (0, 0)),
            pl.BlockSpec((1, H), lambda i: (0, 0)),
        ],
        out_specs=pl.BlockSpec((NC, RBLK, HH), lambda i: (0, i, 0)),
        out_shape=jax.ShapeDtypeStruct((NC, N, HH), jnp.float32),
    )(agg2, w, b_row)


def _final_body(agg_ref, w_ref, b_ref, wout_ref, out_ref, acc_ref):
    i = pl.program_id(0)

    @pl.when(i == 0)
    def _():
        acc_ref[...] = jnp.zeros_like(acc_ref)

    a = agg_ref[...]
    w = w_ref[...]
    y = (jnp.dot(a[0], w[:HH, :], preferred_element_type=jnp.float32)
         + jnp.dot(a[1], w[HH:, :], preferred_element_type=jnp.float32)
         + b_ref[...])
    y = jnp.maximum(y, 0.0)
    acc_ref[...] += jnp.sum(y, axis=0, keepdims=True)

    @pl.when(i == NSTEPS - 1)
    def _():
        out_ref[...] = jnp.sum(acc_ref[...] * wout_ref[...]).reshape(1, 1)


def tc_final(agg2, w, b_row, wout_row):
    return pl.pallas_call(
        _final_body,
        grid=(NSTEPS,),
        in_specs=[
            pl.BlockSpec((NC, RBLK, HH), lambda i: (0, i, 0)),
            pl.BlockSpec((H, H), lambda i: (0, 0)),
            pl.BlockSpec((1, H), lambda i: (0, 0)),
            pl.BlockSpec((1, H), lambda i: (0, 0)),
        ],
        out_specs=pl.BlockSpec((1, 1), lambda i: (0, 0)),
        out_shape=jax.ShapeDtypeStruct((1, 1), jnp.float32),
        scratch_shapes=[pltpu.VMEM((1, H), jnp.float32)],
    )(agg2, w, b_row, wout_row)


def kernel(x, edge_index, edge_attr, W_node, W_edge, W_gcn, b_gcn, W_out):
    del edge_attr, W_edge
    d_in = x.shape[1]
    x_pad = jnp.pad(x, ((0, 0), (0, H - d_in)))
    w_pad = jnp.pad(W_node, ((0, H - d_in), (0, 0)))
    src4 = edge_index[0].reshape(NS, EPT // CH, CH)
    dst4 = edge_index[1].reshape(NS, EPT // CH, CH)

    h2 = tc_embed(x_pad, w_pad)
    for i in range(L):
        agg2 = sc_segment_sum(h2, src4, dst4)
        b_row = b_gcn[i].reshape(1, H)
        if i < L - 1:
            h2 = tc_layer(agg2, W_gcn[i], b_row)
        else:
            out = tc_final(agg2, W_gcn[i], b_row, W_out.reshape(1, H))
    return out


# speedup vs baseline: 13.5846x; 1.1808x over previous
import functools

import jax
import jax.numpy as jnp
from jax import lax
from jax.experimental import pallas as pl
from jax.experimental.pallas import tpu as pltpu
from jax.experimental.pallas import tpu_sc as plsc

N = 50000
E = 800000
H = 64
HH = 32
L = 4

NC = 2
NS = 16

CH = 80
CPS = 5
EPT = E // NS
NCHUNK = EPT // CH
NSUP = NCHUNK // CPS
KR = 10
DP = 5
ZCH = 80
NZCH = N // ZCH
CPCH = 1000
NCP = N // CPCH


def _sc_segment_sum_body(h2, src4, dst4, out, agg, rows, sidx, didx, zrows,
                         isem, gsem, ssem):
    cid = lax.axis_index("c")
    sid = lax.axis_index("s")

    pltpu.async_copy(src4.at[sid].at[pl.ds(0, CPS)], sidx.at[0], isem.at[0])
    pltpu.async_copy(dst4.at[sid].at[pl.ds(0, CPS)], didx.at[0], isem.at[0])

    def zero_row(r, _):
        z = jnp.zeros((16,), jnp.float32)
        zrows[r, pl.ds(0, 16)] = z
        zrows[r, pl.ds(16, 16)] = z
        return 0

    lax.fori_loop(0, ZCH, zero_row, 0)

    def zero_dma(j, _):
        idx = sid + j * NS

        @pl.when(idx < NZCH)
        def _():
            pltpu.sync_copy(zrows, agg.at[pl.ds(idx * ZCH, ZCH)])

        return 0

    lax.fori_loop(0, (NZCH + NS - 1) // NS, zero_dma, 0)

    plsc.subcore_barrier()

    hhalf = h2.at[cid]

    def wait_gather(j):
        pltpu.make_async_copy(hhalf.at[pl.ds(0, CH)], rows.at[j], gsem.at[j]).wait()

    def wait_scatter(j):
        pltpu.make_async_copy(rows.at[j], agg.at[pl.ds(0, CH)], ssem.at[j]).wait()

    def pipeline_step(c, _):
        @pl.when(c < NCHUNK)
        def _():
            sup = c // CPS
            j = lax.rem(c, CPS)
            tb = lax.rem(sup, 3)

            @pl.when(j == 0)
            def _():
                @pl.when(sup < NSUP - 1)
                def _():
                    tn = lax.rem(sup + 1, 3)
                    base = (sup + 1) * CPS
                    pltpu.async_copy(src4.at[sid].at[pl.ds(base, CPS)],
                                     sidx.at[tn], isem.at[tn])
                    pltpu.async_copy(dst4.at[sid].at[pl.ds(base, CPS)],
                                     didx.at[tn], isem.at[tn])

                pltpu.make_async_copy(src4.at[sid].at[pl.ds(0, CPS)],
                                      sidx.at[tb], isem.at[tb]).wait()
                pltpu.make_async_copy(dst4.at[sid].at[pl.ds(0, CPS)],
                                      didx.at[tb], isem.at[tb]).wait()

            b = lax.rem(c, KR)

            @pl.when(c >= KR)
            def _():
                wait_scatter(b)

            pltpu.async_copy(hhalf.at[sidx.at[tb, j]], rows.at[b], gsem.at[b])

        @pl.when(c >= DP)
        def _():
            cc = c - DP
            sup2 = cc // CPS
            j2 = lax.rem(cc, CPS)
            tb2 = lax.rem(sup2, 3)
            b2 = lax.rem(cc, KR)
            wait_gather(b2)
            pltpu.async_copy(rows.at[b2], agg.at[didx.at[tb2, j2]],
                             ssem.at[b2], add=True)

        return 0

    lax.fori_loop(0, NCHUNK + DP, pipeline_step, 0)

    def drain(j, _):
        wait_scatter(j)
        return 0

    lax.fori_loop(0, KR, drain, 0)

    plsc.subcore_barrier()

    def copy_out(j, _):
        idx = sid + j * NS

        @pl.when(idx < NCP)
        def _():
            pltpu.sync_copy(agg.at[pl.ds(idx * CPCH, CPCH)],
                            out.at[cid].at[pl.ds(idx * CPCH, CPCH)])

        return 0

    lax.fori_loop(0, (NCP + NS - 1) // NS, copy_out, 0)


_sc_segment_sum = functools.partial(
    pl.kernel,
    out_type=jax.ShapeDtypeStruct((NC, N, HH), jnp.float32),
    mesh=plsc.VectorSubcoreMesh(core_axis_name="c", subcore_axis_name="s",
                                num_cores=NC, num_subcores=NS),
    scratch_types=[
        pltpu.VMEM_SHARED((N, HH), jnp.float32),
        pltpu.VMEM((KR, CH, HH), jnp.float32),
        pltpu.VMEM((3, CPS, CH), jnp.int32),
        pltpu.VMEM((3, CPS, CH), jnp.int32),
        pltpu.VMEM((ZCH, HH), jnp.float32),
        pltpu.SemaphoreType.DMA((3,)),
        pltpu.SemaphoreType.DMA((KR,)),
        pltpu.SemaphoreType.DMA((KR,)),
    ],
    compiler_params=pltpu.CompilerParams(use_tc_tiling_on_sc=False),
)(_sc_segment_sum_body)


def sc_segment_sum(h2, src4, dst4):
    return _sc_segment_sum(h2, src4, dst4)


RBLK = 2000
NSTEPS = N // RBLK


def _embed_body(x_ref, w_ref, out_ref):
    y = jnp.dot(x_ref[...], w_ref[...], preferred_element_type=jnp.float32)
    out_ref[0] = y[:, :HH]
    out_ref[1] = y[:, HH:]


def tc_embed(x_pad, w_pad):
    return pl.pallas_call(
        _embed_body,
        grid=(NSTEPS,),
        in_specs=[
            pl.BlockSpec((RBLK, H), lambda i: (i, 0)),
            pl.BlockSpec((H, H), lambda i: (0, 0)),
        ],
        out_specs=pl.BlockSpec((NC, RBLK, HH), lambda i: (0, i, 0)),
        out_shape=jax.ShapeDtypeStruct((NC, N, HH), jnp.float32),
    )(x_pad, w_pad)


def _layer_body(agg_ref, w_ref, b_ref, out_ref):
    a = agg_ref[...]
    w = w_ref[...]
    y = (jnp.dot(a[0], w[:HH, :], preferred_element_type=jnp.float32)
         + jnp.dot(a[1], w[HH:, :], preferred_element_type=jnp.float32)
         + b_ref[...])
    y = jnp.maximum(y, 0.0)
    out_ref[0] = y[:, :HH]
    out_ref[1] = y[:, HH:]


def tc_layer(agg2, w, b_row):
    return pl.pallas_call(
        _layer_body,
        grid=(NSTEPS,),
        in_specs=[
            pl.BlockSpec((NC, RBLK, HH), lambda i: (0, i, 0)),
            pl.BlockSpec((H, H), lambda i: (0, 0)),
            pl.BlockSpec((1, H), lambda i: (0, 0)),
        ],
        out_specs=pl.BlockSpec((NC, RBLK, HH), lambda i: (0, i, 0)),
        out_shape=jax.ShapeDtypeStruct((NC, N, HH), jnp.float32),
    )(agg2, w, b_row)


def _final_body(agg_ref, w_ref, b_ref, wout_ref, out_ref, acc_ref):
    i = pl.program_id(0)

    @pl.when(i == 0)
    def _():
        acc_ref[...] = jnp.zeros_like(acc_ref)

    a = agg_ref[...]
    w = w_ref[...]
    y = (jnp.dot(a[0], w[:HH, :], preferred_element_type=jnp.float32)
         + jnp.dot(a[1], w[HH:, :], preferred_element_type=jnp.float32)
         + b_ref[...])
    y = jnp.maximum(y, 0.0)
    acc_ref[...] += jnp.sum(y, axis=0, keepdims=True)

    @pl.when(i == NSTEPS - 1)
    def _():
        out_ref[...] = jnp.sum(acc_ref[...] * wout_ref[...]).reshape(1, 1)


def tc_final(agg2, w, b_row, wout_row):
    return pl.pallas_call(
        _final_body,
        grid=(NSTEPS,),
        in_specs=[
            pl.BlockSpec((NC, RBLK, HH), lambda i: (0, i, 0)),
            pl.BlockSpec((H, H), lambda i: (0, 0)),
            pl.BlockSpec((1, H), lambda i: (0, 0)),
            pl.BlockSpec((1, H), lambda i: (0, 0)),
        ],
        out_specs=pl.BlockSpec((1, 1), lambda i: (0, 0)),
        out_shape=jax.ShapeDtypeStruct((1, 1), jnp.float32),
        scratch_shapes=[pltpu.VMEM((1, H), jnp.float32)],
    )(agg2, w, b_row, wout_row)


def kernel(x, edge_index, edge_attr, W_node, W_edge, W_gcn, b_gcn, W_out):
    del edge_attr, W_edge
    d_in = x.shape[1]
    x_pad = jnp.pad(x, ((0, 0), (0, H - d_in)))
    w_pad = jnp.pad(W_node, ((0, H - d_in), (0, 0)))
    src4 = edge_index[0].reshape(NS, EPT // CH, CH)
    dst4 = edge_index[1].reshape(NS, EPT // CH, CH)

    h2 = tc_embed(x_pad, w_pad)
    for i in range(L):
        agg2 = sc_segment_sum(h2, src4, dst4)
        b_row = b_gcn[i].reshape(1, H)
        if i < L - 1:
            h2 = tc_layer(agg2, W_gcn[i], b_row)
        else:
            out = tc_final(agg2, W_gcn[i], b_row, W_out.reshape(1, H))
    return out


# speedup vs baseline: 13.6865x; 1.0075x over previous
import functools

import jax
import jax.numpy as jnp
from jax import lax
from jax.experimental import pallas as pl
from jax.experimental.pallas import tpu as pltpu
from jax.experimental.pallas import tpu_sc as plsc

N = 50000
E = 800000
H = 64
HH = 32
L = 4

NC = 2
NS = 16

CH = 80
CPS = 5
EPT = E // NS
NCHUNK = EPT // CH
NSUP = NCHUNK // CPS
KR = 10
DP = 5
ZCH = 80
NZCH = N // ZCH
CPCH = 1000
NCP = N // CPCH


def _sc_segment_sum_body(h2, src4, dst4, out, agg, rows, sidx, didx, zrows,
                         isem, gsem, ssem, zsem):
    cid = lax.axis_index("c")
    sid = lax.axis_index("s")

    pltpu.async_copy(src4.at[sid].at[pl.ds(0, CPS)], sidx.at[0], isem.at[0])
    pltpu.async_copy(dst4.at[sid].at[pl.ds(0, CPS)], didx.at[0], isem.at[0])

    def zero_row(r, _):
        z = jnp.zeros((16,), jnp.float32)
        zrows[r, pl.ds(0, 16)] = z
        zrows[r, pl.ds(16, 16)] = z
        return 0

    lax.fori_loop(0, ZCH, zero_row, 0)

    def zero_dma(j, _):
        idx = sid + j * NS

        @pl.when(idx < NZCH)
        def _():
            pltpu.async_copy(zrows, agg.at[pl.ds(idx * ZCH, ZCH)], zsem)

        return 0

    nz = (NZCH + NS - 1) // NS
    lax.fori_loop(0, nz, zero_dma, 0)

    def zero_drain(j, _):
        idx = sid + j * NS

        @pl.when(idx < NZCH)
        def _():
            pltpu.make_async_copy(zrows, agg.at[pl.ds(0, ZCH)], zsem).wait()

        return 0

    lax.fori_loop(0, nz, zero_drain, 0)

    plsc.subcore_barrier()

    hhalf = h2.at[cid]

    def wait_gather(j):
        pltpu.make_async_copy(hhalf.at[pl.ds(0, CH)], rows.at[j], gsem.at[j]).wait()

    def wait_scatter(j):
        pltpu.make_async_copy(rows.at[j], agg.at[pl.ds(0, CH)], ssem.at[j]).wait()

    def pipeline_step(c, _):
        @pl.when(c < NCHUNK)
        def _():
            sup = c // CPS
            j = lax.rem(c, CPS)
            tb = lax.rem(sup, 3)

            @pl.when(j == 0)
            def _():
                @pl.when(sup < NSUP - 1)
                def _():
                    tn = lax.rem(sup + 1, 3)
                    base = (sup + 1) * CPS
                    pltpu.async_copy(src4.at[sid].at[pl.ds(base, CPS)],
                                     sidx.at[tn], isem.at[tn])
                    pltpu.async_copy(dst4.at[sid].at[pl.ds(base, CPS)],
                                     didx.at[tn], isem.at[tn])

                pltpu.make_async_copy(src4.at[sid].at[pl.ds(0, CPS)],
                                      sidx.at[tb], isem.at[tb]).wait()
                pltpu.make_async_copy(dst4.at[sid].at[pl.ds(0, CPS)],
                                      didx.at[tb], isem.at[tb]).wait()

            b = lax.rem(c, KR)

            @pl.when(c >= KR)
            def _():
                wait_scatter(b)

            pltpu.async_copy(hhalf.at[sidx.at[tb, j]], rows.at[b], gsem.at[b])

        @pl.when(c >= DP)
        def _():
            cc = c - DP
            sup2 = cc // CPS
            j2 = lax.rem(cc, CPS)
            tb2 = lax.rem(sup2, 3)
            b2 = lax.rem(cc, KR)
            wait_gather(b2)
            pltpu.async_copy(rows.at[b2], agg.at[didx.at[tb2, j2]],
                             ssem.at[b2], add=True)

        return 0

    lax.fori_loop(0, NCHUNK + DP, pipeline_step, 0)

    def drain(j, _):
        wait_scatter(j)
        return 0

    lax.fori_loop(0, KR, drain, 0)

    plsc.subcore_barrier()

    def copy_out(j, _):
        idx = sid + j * NS

        @pl.when(idx < NCP)
        def _():
            pltpu.async_copy(agg.at[pl.ds(idx * CPCH, CPCH)],
                             out.at[cid].at[pl.ds(idx * CPCH, CPCH)], zsem)

        return 0

    ncp = (NCP + NS - 1) // NS
    lax.fori_loop(0, ncp, copy_out, 0)

    def copy_drain(j, _):
        idx = sid + j * NS

        @pl.when(idx < NCP)
        def _():
            pltpu.make_async_copy(agg.at[pl.ds(0, CPCH)],
                                  out.at[cid].at[pl.ds(0, CPCH)], zsem).wait()

        return 0

    lax.fori_loop(0, ncp, copy_drain, 0)


_sc_segment_sum = functools.partial(
    pl.kernel,
    out_type=jax.ShapeDtypeStruct((NC, N, HH), jnp.float32),
    mesh=plsc.VectorSubcoreMesh(core_axis_name="c", subcore_axis_name="s",
                                num_cores=NC, num_subcores=NS),
    scratch_types=[
        pltpu.VMEM_SHARED((N, HH), jnp.float32),
        pltpu.VMEM((KR, CH, HH), jnp.float32),
        pltpu.VMEM((3, CPS, CH), jnp.int32),
        pltpu.VMEM((3, CPS, CH), jnp.int32),
        pltpu.VMEM((ZCH, HH), jnp.float32),
        pltpu.SemaphoreType.DMA((3,)),
        pltpu.SemaphoreType.DMA((KR,)),
        pltpu.SemaphoreType.DMA((KR,)),
        pltpu.SemaphoreType.DMA,
    ],
    compiler_params=pltpu.CompilerParams(use_tc_tiling_on_sc=False),
)(_sc_segment_sum_body)


def sc_segment_sum(h2, src4, dst4):
    return _sc_segment_sum(h2, src4, dst4)


RBLK = 2000
NSTEPS = N // RBLK


def _embed_body(x_ref, w_ref, out_ref):
    y = jnp.dot(x_ref[...], w_ref[...], preferred_element_type=jnp.float32)
    out_ref[0] = y[:, :HH]
    out_ref[1] = y[:, HH:]


def tc_embed(x_pad, w_pad):
    return pl.pallas_call(
        _embed_body,
        grid=(NSTEPS,),
        in_specs=[
            pl.BlockSpec((RBLK, H), lambda i: (i, 0)),
            pl.BlockSpec((H, H), lambda i: (0, 0)),
        ],
        out_specs=pl.BlockSpec((NC, RBLK, HH), lambda i: (0, i, 0)),
        out_shape=jax.ShapeDtypeStruct((NC, N, HH), jnp.float32),
    )(x_pad, w_pad)


def _layer_body(agg_ref, w_ref, b_ref, out_ref):
    a = agg_ref[...]
    w = w_ref[...]
    y = (jnp.dot(a[0], w[:HH, :], preferred_element_type=jnp.float32)
         + jnp.dot(a[1], w[HH:, :], preferred_element_type=jnp.float32)
         + b_ref[...])
    y = jnp.maximum(y, 0.0)
    out_ref[0] = y[:, :HH]
    out_ref[1] = y[:, HH:]


def tc_layer(agg2, w, b_row):
    return pl.pallas_call(
        _layer_body,
        grid=(NSTEPS,),
        in_specs=[
            pl.BlockSpec((NC, RBLK, HH), lambda i: (0, i, 0)),
            pl.BlockSpec((H, H), lambda i: (0, 0)),
            pl.BlockSpec((1, H), lambda i: (0, 0)),
        ],
        out_specs=pl.BlockSpec((NC, RBLK, HH), lambda i: (0, i, 0)),
        out_shape=jax.ShapeDtypeStruct((NC, N, HH), jnp.float32),
    )(agg2, w, b_row)


def _final_body(agg_ref, w_ref, b_ref, wout_ref, out_ref, acc_ref):
    i = pl.program_id(0)

    @pl.when(i == 0)
    def _():
        acc_ref[...] = jnp.zeros_like(acc_ref)

    a = agg_ref[...]
    w = w_ref[...]
    y = (jnp.dot(a[0], w[:HH, :], preferred_element_type=jnp.float32)
         + jnp.dot(a[1], w[HH:, :], preferred_element_type=jnp.float32)
         + b_ref[...])
    y = jnp.maximum(y, 0.0)
    acc_ref[...] += jnp.sum(y, axis=0, keepdims=True)

    @pl.when(i == NSTEPS - 1)
    def _():
        out_ref[...] = jnp.sum(acc_ref[...] * wout_ref[...]).reshape(1, 1)


def tc_final(agg2, w, b_row, wout_row):
    return pl.pallas_call(
        _final_body,
        grid=(NSTEPS,),
        in_specs=[
            pl.BlockSpec((NC, RBLK, HH), lambda i: (0, i, 0)),
            pl.BlockSpec((H, H), lambda i: (0, 0)),
            pl.BlockSpec((1, H), lambda i: (0, 0)),
            pl.BlockSpec((1, H), lambda i: (0, 0)),
        ],
        out_specs=pl.BlockSpec((1, 1), lambda i: (0, 0)),
        out_shape=jax.ShapeDtypeStruct((1, 1), jnp.float32),
        scratch_shapes=[pltpu.VMEM((1, H), jnp.float32)],
    )(agg2, w, b_row, wout_row)


def kernel(x, edge_index, edge_attr, W_node, W_edge, W_gcn, b_gcn, W_out):
    del edge_attr, W_edge
    d_in = x.shape[1]
    x_pad = jnp.pad(x, ((0, 0), (0, H - d_in)))
    w_pad = jnp.pad(W_node, ((0, H - d_in), (0, 0)))
    src4 = edge_index[0].reshape(NS, EPT // CH, CH)
    dst4 = edge_index[1].reshape(NS, EPT // CH, CH)

    h2 = tc_embed(x_pad, w_pad)
    for i in range(L):
        agg2 = sc_segment_sum(h2, src4, dst4)
        b_row = b_gcn[i].reshape(1, H)
        if i < L - 1:
            h2 = tc_layer(agg2, W_gcn[i], b_row)
        else:
            out = tc_final(agg2, W_gcn[i], b_row, W_out.reshape(1, H))
    return out


# speedup vs baseline: 16.8430x; 1.2306x over previous
import functools

import jax
import jax.numpy as jnp
from jax import lax
from jax.experimental import pallas as pl
from jax.experimental.pallas import tpu as pltpu
from jax.experimental.pallas import tpu_sc as plsc

N = 50000
E = 800000
H = 64
HH = 32
L = 4

NC = 2
NS = 16

CH = 80
CPS = 5
EPT = E // NS
NCHUNK = EPT // CH
NSUP = NCHUNK // CPS
KR = 10
DP = 5
ZCH = 80
NZCH = N // ZCH
CPCH = 1000
NCP = N // CPCH


def _sc_segment_sum_body(h2, src4, dst4, out, agg, rows, sidx, didx, zrows,
                         isem, gsem, ssem, zsem):
    cid = lax.axis_index("c")
    sid = lax.axis_index("s")

    pltpu.async_copy(src4.at[sid].at[pl.ds(0, CPS)], sidx.at[0], isem.at[0])
    pltpu.async_copy(dst4.at[sid].at[pl.ds(0, CPS)], didx.at[0], isem.at[0])

    def zero_row(r, _):
        z = jnp.zeros((16,), jnp.float32)
        zrows[r, pl.ds(0, 16)] = z
        zrows[r, pl.ds(16, 16)] = z
        return 0

    lax.fori_loop(0, ZCH, zero_row, 0)

    def zero_dma(j, _):
        idx = sid + j * NS

        @pl.when(idx < NZCH)
        def _():
            pltpu.async_copy(zrows, agg.at[pl.ds(idx * ZCH, ZCH)], zsem)

        return 0

    nz = (NZCH + NS - 1) // NS
    lax.fori_loop(0, nz, zero_dma, 0)

    def zero_drain(j, _):
        idx = sid + j * NS

        @pl.when(idx < NZCH)
        def _():
            pltpu.make_async_copy(zrows, agg.at[pl.ds(0, ZCH)], zsem).wait()

        return 0

    lax.fori_loop(0, nz, zero_drain, 0)

    plsc.subcore_barrier()

    hhalf = h2.at[cid]

    def wait_gather(j):
        pltpu.make_async_copy(hhalf.at[pl.ds(0, CH)], rows.at[j], gsem.at[j]).wait()

    def wait_scatter(j):
        pltpu.make_async_copy(rows.at[j], agg.at[pl.ds(0, CH)], ssem.at[j]).wait()

    def pipeline_step(c, _):
        @pl.when(c < NCHUNK)
        def _():
            sup = c // CPS
            j = lax.rem(c, CPS)
            tb = lax.rem(sup, 3)

            @pl.when(j == 0)
            def _():
                @pl.when(sup < NSUP - 1)
                def _():
                    tn = lax.rem(sup + 1, 3)
                    base = (sup + 1) * CPS
                    pltpu.async_copy(src4.at[sid].at[pl.ds(base, CPS)],
                                     sidx.at[tn], isem.at[tn])
                    pltpu.async_copy(dst4.at[sid].at[pl.ds(base, CPS)],
                                     didx.at[tn], isem.at[tn])

                pltpu.make_async_copy(src4.at[sid].at[pl.ds(0, CPS)],
                                      sidx.at[tb], isem.at[tb]).wait()
                pltpu.make_async_copy(dst4.at[sid].at[pl.ds(0, CPS)],
                                      didx.at[tb], isem.at[tb]).wait()

            b = lax.rem(c, KR)

            @pl.when(c >= KR)
            def _():
                wait_scatter(b)

            pltpu.async_copy(hhalf.at[sidx.at[tb, j]], rows.at[b], gsem.at[b])

        @pl.when(c >= DP)
        def _():
            cc = c - DP
            sup2 = cc // CPS
            j2 = lax.rem(cc, CPS)
            tb2 = lax.rem(sup2, 3)
            b2 = lax.rem(cc, KR)
            wait_gather(b2)
            pltpu.async_copy(rows.at[b2], agg.at[didx.at[tb2, j2]],
                             ssem.at[b2], add=True)

        return 0

    lax.fori_loop(0, NCHUNK + DP, pipeline_step, 0)

    def drain(j, _):
        wait_scatter(j)
        return 0

    lax.fori_loop(0, KR, drain, 0)

    plsc.subcore_barrier()

    def copy_out(j, _):
        idx = sid + j * NS

        @pl.when(idx < NCP)
        def _():
            pltpu.async_copy(agg.at[pl.ds(idx * CPCH, CPCH)],
                             out.at[cid].at[pl.ds(idx * CPCH, CPCH)], zsem)

        return 0

    ncp = (NCP + NS - 1) // NS
    lax.fori_loop(0, ncp, copy_out, 0)

    def copy_drain(j, _):
        idx = sid + j * NS

        @pl.when(idx < NCP)
        def _():
            pltpu.make_async_copy(agg.at[pl.ds(0, CPCH)],
                                  out.at[cid].at[pl.ds(0, CPCH)], zsem).wait()

        return 0

    lax.fori_loop(0, ncp, copy_drain, 0)


_sc_segment_sum = functools.partial(
    pl.kernel,
    out_type=jax.ShapeDtypeStruct((NC, N, HH), jnp.float32),
    mesh=plsc.VectorSubcoreMesh(core_axis_name="c", subcore_axis_name="s",
                                num_cores=NC, num_subcores=NS),
    scratch_types=[
        pltpu.VMEM_SHARED((N, HH), jnp.float32),
        pltpu.VMEM((KR, CH, HH), jnp.float32),
        pltpu.VMEM((3, CPS, CH), jnp.int32),
        pltpu.VMEM((3, CPS, CH), jnp.int32),
        pltpu.VMEM((ZCH, HH), jnp.float32),
        pltpu.SemaphoreType.DMA((3,)),
        pltpu.SemaphoreType.DMA((KR,)),
        pltpu.SemaphoreType.DMA((KR,)),
        pltpu.SemaphoreType.DMA,
    ],
    compiler_params=pltpu.CompilerParams(use_tc_tiling_on_sc=False),
)(_sc_segment_sum_body)


def sc_segment_sum(h2, src4, dst4):
    return _sc_segment_sum(h2, src4, dst4)


NP4 = N // 4
RBLK = 500
NSTEPS = NP4 // RBLK


def _embed_body(xp_ref, w_ref, out_ref):
    w = w_ref[...]
    xp = xp_ref[0]
    for q in range(4):
        y = jnp.dot(xp[:, 64 * q:64 * q + 64], w,
                    preferred_element_type=jnp.float32)
        out_ref[0, 0, :, 32 * q:32 * q + 32] = y[:, :HH]
        out_ref[1, 0, :, 32 * q:32 * q + 32] = y[:, HH:]


def tc_embed(x_packed, w_pad):
    return pl.pallas_call(
        _embed_body,
        grid=(NSTEPS,),
        in_specs=[
            pl.BlockSpec((1, RBLK, 256), lambda i: (i, 0, 0)),
            pl.BlockSpec((H, H), lambda i: (0, 0)),
        ],
        out_specs=pl.BlockSpec((NC, 1, RBLK, 128), lambda i: (0, i, 0, 0)),
        out_shape=jax.ShapeDtypeStruct((NC, NSTEPS, RBLK, 128), jnp.float32),
    )(x_packed, w_pad)


def _layer_body(agg_ref, w_ref, b_ref, out_ref):
    w = w_ref[...]
    b = b_ref[...]
    p0 = agg_ref[0, 0]
    p1 = agg_ref[1, 0]
    for q in range(4):
        a = jnp.concatenate([p0[:, 32 * q:32 * q + 32],
                             p1[:, 32 * q:32 * q + 32]], axis=1)
        y = jnp.dot(a, w, preferred_element_type=jnp.float32) + b
        y = jnp.maximum(y, 0.0)
        out_ref[0, 0, :, 32 * q:32 * q + 32] = y[:, :HH]
        out_ref[1, 0, :, 32 * q:32 * q + 32] = y[:, HH:]


def tc_layer(agg2, w, b_row):
    return pl.pallas_call(
        _layer_body,
        grid=(NSTEPS,),
        in_specs=[
            pl.BlockSpec((NC, 1, RBLK, 128), lambda i: (0, i, 0, 0)),
            pl.BlockSpec((H, H), lambda i: (0, 0)),
            pl.BlockSpec((1, H), lambda i: (0, 0)),
        ],
        out_specs=pl.BlockSpec((NC, 1, RBLK, 128), lambda i: (0, i, 0, 0)),
        out_shape=jax.ShapeDtypeStruct((NC, NSTEPS, RBLK, 128), jnp.float32),
    )(agg2, w, b_row)


def _final_body(agg_ref, w_ref, b_ref, wout_ref, out_ref, acc_ref):
    i = pl.program_id(0)

    @pl.when(i == 0)
    def _():
        acc_ref[...] = jnp.zeros_like(acc_ref)

    w = w_ref[...]
    b = b_ref[...]
    p0 = agg_ref[0, 0]
    p1 = agg_ref[1, 0]
    for q in range(4):
        a = jnp.concatenate([p0[:, 32 * q:32 * q + 32],
                             p1[:, 32 * q:32 * q + 32]], axis=1)
        y = jnp.dot(a, w, preferred_element_type=jnp.float32) + b
        y = jnp.maximum(y, 0.0)
        acc_ref[...] += jnp.sum(y, axis=0, keepdims=True)

    @pl.when(i == NSTEPS - 1)
    def _():
        out_ref[...] = jnp.sum(acc_ref[...] * wout_ref[...]).reshape(1, 1)


def tc_final(agg2, w, b_row, wout_row):
    return pl.pallas_call(
        _final_body,
        grid=(NSTEPS,),
        in_specs=[
            pl.BlockSpec((NC, 1, RBLK, 128), lambda i: (0, i, 0, 0)),
            pl.BlockSpec((H, H), lambda i: (0, 0)),
            pl.BlockSpec((1, H), lambda i: (0, 0)),
            pl.BlockSpec((1, H), lambda i: (0, 0)),
        ],
        out_specs=pl.BlockSpec((1, 1), lambda i: (0, 0)),
        out_shape=jax.ShapeDtypeStruct((1, 1), jnp.float32),
        scratch_shapes=[pltpu.VMEM((1, H), jnp.float32)],
    )(agg2, w, b_row, wout_row)


def kernel(x, edge_index, edge_attr, W_node, W_edge, W_gcn, b_gcn, W_out):
    del edge_attr, W_edge
    d_in = x.shape[1]
    x_packed = jnp.pad(x, ((0, 0), (0, H - d_in))).reshape(NSTEPS, RBLK, 4 * H)
    w_pad = jnp.pad(W_node, ((0, H - d_in), (0, 0)))
    src4 = edge_index[0].reshape(NS, EPT // CH, CH)
    dst4 = edge_index[1].reshape(NS, EPT // CH, CH)

    h2 = tc_embed(x_packed, w_pad)
    for i in range(L):
        agg2 = sc_segment_sum(h2.reshape(NC, N, HH), src4,
                              dst4).reshape(NC, NSTEPS, RBLK, 128)
        b_row = b_gcn[i].reshape(1, H)
        if i < L - 1:
            h2 = tc_layer(agg2, W_gcn[i], b_row)
        else:
            out = tc_final(agg2, W_gcn[i], b_row, W_out.reshape(1, H))
    return out


# speedup vs baseline: 18.4221x; 1.0938x over previous
import functools

import jax
import jax.numpy as jnp
from jax import lax
from jax.experimental import pallas as pl
from jax.experimental.pallas import tpu as pltpu
from jax.experimental.pallas import tpu_sc as plsc

N = 50000
E = 800000
H = 64
HH = 32
L = 4

NC = 2
NS = 16

CH = 128
NCHG = E // CH
CPB = NCHG // NS
CPX = NCHG - CPB * NS
CPS = 5
KR = 6
DP = 3
ZCH = 50
NZCH = N // ZCH
CPCH = 1000
NCP = N // CPCH


def _sc_segment_sum_body(h2, src2, dst2, out, agg, rows, sidx, didx, zrows,
                         isem, gsem, ssem, zsem):
    cid = lax.axis_index("c")
    sid = lax.axis_index("s")

    cbase = sid * CPB + jnp.minimum(sid, CPX)
    nch = CPB + jnp.where(sid < CPX, 1, 0)

    pltpu.async_copy(src2.at[pl.ds(cbase, CPS)], sidx.at[0], isem.at[0])
    pltpu.async_copy(dst2.at[pl.ds(cbase, CPS)], didx.at[0], isem.at[0])

    def zero_row(r, _):
        z = jnp.zeros((16,), jnp.float32)
        zrows[r, pl.ds(0, 16)] = z
        zrows[r, pl.ds(16, 16)] = z
        return 0

    lax.fori_loop(0, ZCH, zero_row, 0)

    def zero_dma(j, _):
        idx = sid + j * NS

        @pl.when(idx < NZCH)
        def _():
            pltpu.async_copy(zrows, agg.at[pl.ds(idx * ZCH, ZCH)], zsem)

        return 0

    nz = (NZCH + NS - 1) // NS
    lax.fori_loop(0, nz, zero_dma, 0)

    def zero_drain(j, _):
        idx = sid + j * NS

        @pl.when(idx < NZCH)
        def _():
            pltpu.make_async_copy(zrows, agg.at[pl.ds(0, ZCH)], zsem).wait()

        return 0

    lax.fori_loop(0, nz, zero_drain, 0)

    plsc.subcore_barrier()

    hhalf = h2.at[cid]

    def wait_gather(j):
        pltpu.make_async_copy(hhalf.at[pl.ds(0, CH)], rows.at[j],
                              gsem.at[j]).wait()

    def wait_scatter(j):
        pltpu.make_async_copy(rows.at[j], agg.at[pl.ds(0, CH)],
                              ssem.at[j]).wait()

    def pipeline_step(c, _):
        @pl.when(c < nch)
        def _():
            sup = c // CPS
            j = lax.rem(c, CPS)
            tb = lax.rem(sup, 3)

            @pl.when(j == 0)
            def _():
                @pl.when((sup + 1) * CPS < nch)
                def _():
                    tn = lax.rem(sup + 1, 3)
                    base = cbase + (sup + 1) * CPS
                    pltpu.async_copy(src2.at[pl.ds(base, CPS)],
                                     sidx.at[tn], isem.at[tn])
                    pltpu.async_copy(dst2.at[pl.ds(base, CPS)],
                                     didx.at[tn], isem.at[tn])

                pltpu.make_async_copy(src2.at[pl.ds(0, CPS)],
                                      sidx.at[tb], isem.at[tb]).wait()
                pltpu.make_async_copy(dst2.at[pl.ds(0, CPS)],
                                      didx.at[tb], isem.at[tb]).wait()

            b = lax.rem(c, KR)

            @pl.when(c >= KR)
            def _():
                wait_scatter(b)

            pltpu.async_copy(hhalf.at[sidx.at[tb, j]], rows.at[b], gsem.at[b])

        @pl.when(c >= DP)
        def _():
            cc = c - DP
            sup2 = cc // CPS
            j2 = lax.rem(cc, CPS)
            tb2 = lax.rem(sup2, 3)
            b2 = lax.rem(cc, KR)
            wait_gather(b2)
            pltpu.async_copy(rows.at[b2], agg.at[didx.at[tb2, j2]],
                             ssem.at[b2], add=True)

        return 0

    lax.fori_loop(0, nch + DP, pipeline_step, 0)

    def drain(j, _):
        wait_scatter(j)
        return 0

    lax.fori_loop(0, KR, drain, 0)

    plsc.subcore_barrier()

    def copy_out(j, _):
        idx = sid + j * NS

        @pl.when(idx < NCP)
        def _():
            pltpu.async_copy(agg.at[pl.ds(idx * CPCH, CPCH)],
                             out.at[cid].at[pl.ds(idx * CPCH, CPCH)], zsem)

        return 0

    ncp = (NCP + NS - 1) // NS
    lax.fori_loop(0, ncp, copy_out, 0)

    def copy_drain(j, _):
        idx = sid + j * NS

        @pl.when(idx < NCP)
        def _():
            pltpu.make_async_copy(agg.at[pl.ds(0, CPCH)],
                                  out.at[cid].at[pl.ds(0, CPCH)], zsem).wait()

        return 0

    lax.fori_loop(0, ncp, copy_drain, 0)


_sc_segment_sum = functools.partial(
    pl.kernel,
    out_type=jax.ShapeDtypeStruct((NC, N, HH), jnp.float32),
    mesh=plsc.VectorSubcoreMesh(core_axis_name="c", subcore_axis_name="s",
                                num_cores=NC, num_subcores=NS),
    scratch_types=[
        pltpu.VMEM_SHARED((N, HH), jnp.float32),
        pltpu.VMEM((KR, CH, HH), jnp.float32),
        pltpu.VMEM((3, CPS, CH), jnp.int32),
        pltpu.VMEM((3, CPS, CH), jnp.int32),
        pltpu.VMEM((ZCH, HH), jnp.float32),
        pltpu.SemaphoreType.DMA((3,)),
        pltpu.SemaphoreType.DMA((KR,)),
        pltpu.SemaphoreType.DMA((KR,)),
        pltpu.SemaphoreType.DMA,
    ],
    compiler_params=pltpu.CompilerParams(use_tc_tiling_on_sc=False),
)(_sc_segment_sum_body)


def sc_segment_sum(h2, src2, dst2):
    return _sc_segment_sum(h2, src2, dst2)


NP4 = N // 4
RBLK = 500
NSTEPS = NP4 // RBLK


def _embed_body(xp_ref, m_ref, out_ref):
    o = jnp.dot(xp_ref[0], m_ref[...], preferred_element_type=jnp.float32)
    out_ref[0, 0] = o[:, :128]
    out_ref[1, 0] = o[:, 128:]


def tc_embed(x_packed, m_embed):
    return pl.pallas_call(
        _embed_body,
        grid=(NSTEPS,),
        in_specs=[
            pl.BlockSpec((1, RBLK, 256), lambda i: (i, 0, 0)),
            pl.BlockSpec((256, 256), lambda i: (0, 0)),
        ],
        out_specs=pl.BlockSpec((NC, 1, RBLK, 128), lambda i: (0, i, 0, 0)),
        out_shape=jax.ShapeDtypeStruct((NC, NSTEPS, RBLK, 128), jnp.float32),
    )(x_packed, m_embed)


def _layer_body(agg_ref, m0_ref, m1_ref, b_ref, out_ref):
    o = (jnp.dot(agg_ref[0, 0], m0_ref[...], preferred_element_type=jnp.float32)
         + jnp.dot(agg_ref[1, 0], m1_ref[...], preferred_element_type=jnp.float32)
         + b_ref[...])
    o = jnp.maximum(o, 0.0)
    out_ref[0, 0] = o[:, :128]
    out_ref[1, 0] = o[:, 128:]


def tc_layer(agg2, m0, m1, b_packed):
    return pl.pallas_call(
        _layer_body,
        grid=(NSTEPS,),
        in_specs=[
            pl.BlockSpec((NC, 1, RBLK, 128), lambda i: (0, i, 0, 0)),
            pl.BlockSpec((128, 256), lambda i: (0, 0)),
            pl.BlockSpec((128, 256), lambda i: (0, 0)),
            pl.BlockSpec((1, 256), lambda i: (0, 0)),
        ],
        out_specs=pl.BlockSpec((NC, 1, RBLK, 128), lambda i: (0, i, 0, 0)),
        out_shape=jax.ShapeDtypeStruct((NC, NSTEPS, RBLK, 128), jnp.float32),
    )(agg2, m0, m1, b_packed)


def _final_body(agg_ref, m0_ref, m1_ref, b_ref, wout_ref, out_ref, acc_ref):
    i = pl.program_id(0)

    @pl.when(i == 0)
    def _():
        acc_ref[...] = jnp.zeros_like(acc_ref)

    o = (jnp.dot(agg_ref[0, 0], m0_ref[...], preferred_element_type=jnp.float32)
         + jnp.dot(agg_ref[1, 0], m1_ref[...], preferred_element_type=jnp.float32)
         + b_ref[...])
    o = jnp.maximum(o, 0.0)
    acc_ref[...] += jnp.sum(o, axis=0, keepdims=True)

    @pl.when(i == NSTEPS - 1)
    def _():
        acc = acc_ref[...]
        s0 = acc[:, 0:32] + acc[:, 32:64] + acc[:, 64:96] + acc[:, 96:128]
        s1 = (acc[:, 128:160] + acc[:, 160:192] + acc[:, 192:224]
              + acc[:, 224:256])
        w = wout_ref[...]
        tot = jnp.sum(s0 * w[:, :HH]) + jnp.sum(s1 * w[:, HH:])
        out_ref[...] = tot.reshape(1, 1)


def tc_final(agg2, m0, m1, b_packed, wout_row):
    return pl.pallas_call(
        _final_body,
        grid=(NSTEPS,),
        in_specs=[
            pl.BlockSpec((NC, 1, RBLK, 128), lambda i: (0, i, 0, 0)),
            pl.BlockSpec((128, 256), lambda i: (0, 0)),
            pl.BlockSpec((128, 256), lambda i: (0, 0)),
            pl.BlockSpec((1, 256), lambda i: (0, 0)),
            pl.BlockSpec((1, H), lambda i: (0, 0)),
        ],
        out_specs=pl.BlockSpec((1, 1), lambda i: (0, 0)),
        out_shape=jax.ShapeDtypeStruct((1, 1), jnp.float32),
        scratch_shapes=[pltpu.VMEM((1, 256), jnp.float32)],
    )(agg2, m0, m1, b_packed, wout_row)


def _expand_gcn_w(w):
    wr = w.reshape(2, 32, 2, 32)
    m = jnp.einsum('cfdj,qp->cqfdpj', wr, jnp.eye(4, dtype=w.dtype))
    return m.reshape(256, 256)


def _expand_embed_w(w):
    wr = w.reshape(64, 2, 32)
    m = jnp.einsum('dcj,qp->qdcpj', wr, jnp.eye(4, dtype=w.dtype))
    return m.reshape(256, 256)


def kernel(x, edge_index, edge_attr, W_node, W_edge, W_gcn, b_gcn, W_out):
    del edge_attr, W_edge
    d_in = x.shape[1]
    x_packed = jnp.pad(x, ((0, 0), (0, H - d_in))).reshape(NSTEPS, RBLK, 4 * H)
    m_embed = _expand_embed_w(jnp.pad(W_node, ((0, H - d_in), (0, 0))))
    src2 = edge_index[0].reshape(NCHG, CH)
    dst2 = edge_index[1].reshape(NCHG, CH)

    h2 = tc_embed(x_packed, m_embed)
    for i in range(L):
        agg2 = sc_segment_sum(h2.reshape(NC, N, HH), src2,
                              dst2).reshape(NC, NSTEPS, RBLK, 128)
        m = _expand_gcn_w(W_gcn[i])
        m0, m1 = m[:128], m[128:]
        b_packed = jnp.tile(b_gcn[i].reshape(2, 1, 32),
                            (1, 4, 1)).reshape(1, 256)
        if i < L - 1:
            h2 = tc_layer(agg2, m0, m1, b_packed)
        else:
            out = tc_final(agg2, m0, m1, b_packed, W_out.reshape(1, H))
    return out
